# baseline (speedup 1.0000x reference)
import functools

import jax
import jax.numpy as jnp
from jax import lax
from jax.experimental import pallas as pl
from jax.experimental.pallas import tpu as pltpu
from jax.experimental.pallas import tpu_sc as plsc

_N = 10000
_E = 320000
_D = 128
_H = 128
_A = 64

_NC = 2
_NS = 16
_TILES = _NC * _NS
_CH = 128
_NCHUNK = _E // _CH
_CPT = _NCHUNK // _TILES
_XTRA = _NCHUNK - _CPT * _TILES
_EPT = _CPT * _CH
_NB = 2
_NP = 10240
_RPT = _NP // _NS


def _tc_y_body(x_ref, wmsg_ref, y_ref):
    y_ref[...] = jnp.maximum(jnp.dot(x_ref[...], wmsg_ref[...]), 0.0)


_tc_y = pl.pallas_call(
    _tc_y_body,
    out_shape=jax.ShapeDtypeStruct((_N, _H), jnp.float32),
)


def _tc_xu_body(x_ref, wupd_ref, xu_ref):
    xu_ref[...] = jnp.dot(x_ref[...], wupd_ref[: _D])


_tc_xu = pl.pallas_call(
    _tc_xu_body,
    out_shape=jax.ShapeDtypeStruct((_N, _H), jnp.float32),
)


_sc_mesh = plsc.VectorSubcoreMesh(core_axis_name="c", subcore_axis_name="s")


@functools.partial(
    pl.kernel,
    mesh=_sc_mesh,
    out_type=jax.ShapeDtypeStruct((_NC, _NP, _H), jnp.float32),
    scratch_types=[
        pltpu.VMEM((_EPT + _CH,), jnp.int32),
        [pltpu.VMEM((_CH,), jnp.int32) for _ in range(_NB)],
        [pltpu.VMEM((_CH, _H), jnp.float32) for _ in range(_NB)],
        pltpu.VMEM_SHARED((_NP, _H), jnp.float32),
        pltpu.SemaphoreType.DMA,
        pltpu.SemaphoreType.DMA,
    ],
)
def _sc_agg(y_hbm, src_hbm, dst_hbm, z_hbm, out_hbm,
            src_v, dstb, rows, agg_sh, gsem, dsem):
    c = lax.axis_index("c")
    s = lax.axis_index("s")
    t = c * _NS + s
    ebase = t * _EPT
    xbase = _TILES * _EPT + t * _CH

    pltpu.sync_copy(src_hbm.at[pl.ds(ebase, _EPT)],
                    src_v.at[pl.ds(0, _EPT)])

    @pl.when(t < _XTRA)
    def _():
        pltpu.sync_copy(src_hbm.at[pl.ds(xbase, _CH)],
                        src_v.at[pl.ds(_EPT, _CH)])

    for b in range(_NB):
        pltpu.async_copy(dst_hbm.at[pl.ds(ebase + b * _CH, _CH)],
                         dstb[b], dsem)
        pltpu.async_copy(
            y_hbm.at[src_v.at[pl.ds(b * _CH, _CH)]], rows[b], gsem)

    row0 = s * _RPT
    pltpu.sync_copy(z_hbm, agg_sh.at[pl.ds(row0, _RPT)])
    plsc.subcore_barrier()

    def step(i, b, prefetch):
        pltpu.make_async_copy(
            y_hbm.at[src_v.at[pl.ds(i * _CH, _CH)]], rows[b], gsem).wait()
        pltpu.make_async_copy(
            dst_hbm.at[pl.ds(0, _CH)], dstb[b], dsem).wait()
        pltpu.sync_copy(rows[b], agg_sh.at[dstb[b]], add=True)
        if prefetch:
            pltpu.async_copy(
                dst_hbm.at[pl.ds(ebase + (i + _NB) * _CH, _CH)],
                dstb[b], dsem)
            pltpu.async_copy(
                y_hbm.at[src_v.at[pl.ds((i + _NB) * _CH, _CH)]],
                rows[b], gsem)

    def body(j, carry):
        for b in range(_NB):
            step(j * _NB + b, b, True)
        return carry

    lax.fori_loop(0, (_CPT - _NB) // _NB, body, 0)
    for b in range(_NB):
        step(_CPT - _NB + b, b, False)

    @pl.when(t < _XTRA)
    def _():
        pltpu.sync_copy(dst_hbm.at[pl.ds(xbase, _CH)], dstb[0])
        pltpu.async_copy(
            y_hbm.at[src_v.at[pl.ds(_EPT, _CH)]], rows[0], gsem).wait()
        pltpu.sync_copy(rows[0], agg_sh.at[dstb[0]], add=True)

    plsc.subcore_barrier()
    pltpu.sync_copy(agg_sh.at[pl.ds(row0, _RPT)],
                    out_hbm.at[c, pl.ds(row0, _RPT)])


def _tc_post_body(xu_ref, aggs_ref, wupd_ref, w1_ref, w2t_ref, out_ref):
    agg = aggs_ref[0, : _N] + aggs_ref[1, : _N]
    nf = jnp.maximum(xu_ref[...] + jnp.dot(agg, wupd_ref[_D:]), 0.0)
    h1 = jnp.maximum(jnp.dot(nf, w1_ref[...]), 0.0)
    logits = jnp.sum(h1 * w2t_ref[...], axis=1, keepdims=True)
    m = jnp.max(logits)
    e = jnp.exp(logits - m)
    out_ref[...] = e / jnp.sum(e)


_tc_post = pl.pallas_call(
    _tc_post_body,
    out_shape=jax.ShapeDtypeStruct((_N, 1), jnp.float32),
)


def kernel(x, edge_index, W_msg, W_upd, w_actor1, w_actor2):
    y = _tc_y(x, W_msg)
    xu = _tc_xu(x, W_upd)
    z = jnp.zeros((_RPT, _H), jnp.float32)
    aggs = _sc_agg(y, edge_index[0], edge_index[1], z)
    probs = _tc_post(xu, aggs, W_upd, w_actor1, w_actor2.T)
    return probs[:, 0]


# speedup vs baseline: 1.1132x; 1.1132x over previous
import functools

import jax
import jax.numpy as jnp
from jax import lax
from jax.experimental import pallas as pl
from jax.experimental.pallas import tpu as pltpu
from jax.experimental.pallas import tpu_sc as plsc

_N = 10000
_E = 320000
_D = 128
_H = 128
_A = 64

_NC = 2
_NS = 16
_TILES = _NC * _NS
_CH = 128
_NCHUNK = _E // _CH
_CPT = _NCHUNK // _TILES
_XTRA = _NCHUNK - _CPT * _TILES
_EPT = _CPT * _CH
_NB = 2
_NP = 10240
_RPT = _NP // _NS


def _tc_y_body(x_ref, wmsg_ref, y_ref):
    y_ref[...] = jnp.maximum(jnp.dot(x_ref[...], wmsg_ref[...]), 0.0)


_tc_y = pl.pallas_call(
    _tc_y_body,
    out_shape=jax.ShapeDtypeStruct((_N, _H), jnp.float32),
)


def _tc_xu_body(x_ref, wupd_ref, xu_ref):
    xu_ref[...] = jnp.dot(x_ref[...], wupd_ref[: _D])


_tc_xu = pl.pallas_call(
    _tc_xu_body,
    out_shape=jax.ShapeDtypeStruct((_N, _H), jnp.float32),
)


_sc_mesh = plsc.VectorSubcoreMesh(core_axis_name="c", subcore_axis_name="s")


_NI = 4


@functools.partial(
    pl.kernel,
    mesh=_sc_mesh,
    out_type=jax.ShapeDtypeStruct((_NC, _NP, _H), jnp.float32),
    scratch_types=[
        [pltpu.VMEM((2, _CH), jnp.int32) for _ in range(_NI)],
        [pltpu.VMEM((_CH, _H), jnp.float32) for _ in range(_NB)],
        pltpu.VMEM_SHARED((_NP, _H), jnp.float32),
        pltpu.SemaphoreType.DMA,
        pltpu.SemaphoreType.DMA,
    ],
)
def _sc_agg(y_hbm, ei_hbm, z_hbm, out_hbm,
            idxb, rows, agg_sh, gsem, isem):
    c = lax.axis_index("c")
    s = lax.axis_index("s")
    t = c * _NS + s
    ebase = t * _EPT
    xbase = _TILES * _EPT + t * _CH

    def load_idx(off, k):
        pltpu.async_copy(ei_hbm.at[pl.ds(0, 2), pl.ds(off, _CH)],
                         idxb[k], isem)

    def drain_idx(k):
        pltpu.make_async_copy(
            ei_hbm.at[pl.ds(0, 2), pl.ds(0, _CH)], idxb[k], isem).wait()

    for k in range(_NI):
        load_idx(ebase + k * _CH, k)
    for b in range(_NB):
        drain_idx(b)
        pltpu.async_copy(y_hbm.at[idxb[b].at[0]], rows[b], gsem)

    row0 = s * _RPT
    pltpu.sync_copy(z_hbm, agg_sh.at[pl.ds(row0, _RPT)])
    plsc.subcore_barrier()

    def step(i, iv, load, feed):
        b = i % _NB
        k = i % _NI
        pltpu.make_async_copy(
            y_hbm.at[idxb[k].at[0]], rows[b], gsem).wait()
        pltpu.sync_copy(rows[b], agg_sh.at[idxb[k].at[1]], add=True)
        if load:
            load_idx(ebase + (iv + _NI) * _CH, k)
        if feed:
            kg = (i + _NB) % _NI
            drain_idx(kg)
            pltpu.async_copy(y_hbm.at[idxb[kg].at[0]], rows[b], gsem)

    _MAIN = (_CPT - _NI - 2) // _NI * _NI

    def body(j, carry):
        for u in range(_NI):
            step(u, j * _NI + u, True, True)
        return carry

    lax.fori_loop(0, _MAIN // _NI, body, 0)
    for i in range(_MAIN, _CPT):
        step(i, i, i + _NI < _CPT, i + _NB < _CPT)

    @pl.when(t < _XTRA)
    def _():
        load_idx(xbase, 0)
        drain_idx(0)
        pltpu.async_copy(y_hbm.at[idxb[0].at[0]], rows[0], gsem).wait()
        pltpu.sync_copy(rows[0], agg_sh.at[idxb[0].at[1]], add=True)

    plsc.subcore_barrier()
    pltpu.sync_copy(agg_sh.at[pl.ds(row0, _RPT)],
                    out_hbm.at[c, pl.ds(row0, _RPT)])


def _tc_post_body(xu_ref, aggs_ref, wupd_ref, w1_ref, w2t_ref, out_ref):
    agg = aggs_ref[0, : _N] + aggs_ref[1, : _N]
    nf = jnp.maximum(xu_ref[...] + jnp.dot(agg, wupd_ref[_D:]), 0.0)
    h1 = jnp.maximum(jnp.dot(nf, w1_ref[...]), 0.0)
    logits = jnp.sum(h1 * w2t_ref[...], axis=1, keepdims=True)
    m = jnp.max(logits)
    e = jnp.exp(logits - m)
    out_ref[...] = e / jnp.sum(e)


_tc_post = pl.pallas_call(
    _tc_post_body,
    out_shape=jax.ShapeDtypeStruct((_N, 1), jnp.float32),
)


def kernel(x, edge_index, W_msg, W_upd, w_actor1, w_actor2):
    y = _tc_y(x, W_msg)
    xu = _tc_xu(x, W_upd)
    z = jnp.zeros((_RPT, _H), jnp.float32)
    aggs = _sc_agg(y, edge_index, z)
    probs = _tc_post(xu, aggs, W_upd, w_actor1, w_actor2.T)
    return probs[:, 0]


# speedup vs baseline: 1.1527x; 1.0354x over previous
import functools

import jax
import jax.numpy as jnp
from jax import lax
from jax.experimental import pallas as pl
from jax.experimental.pallas import tpu as pltpu
from jax.experimental.pallas import tpu_sc as plsc

_N = 10000
_E = 320000
_D = 128
_H = 128
_A = 64

_NC = 2
_NS = 16
_TILES = _NC * _NS
_CH = 128
_NCHUNK = _E // _CH
_CPT = _NCHUNK // _TILES
_XTRA = _NCHUNK - _CPT * _TILES
_EPT = _CPT * _CH
_NB = 2
_NP = 10240
_RPT = _NP // _NS
_ZR = 32


def _tc_y_body(x_ref, wmsg_ref, y_ref):
    y_ref[...] = jnp.maximum(jnp.dot(x_ref[...], wmsg_ref[...]), 0.0)


_tc_y = pl.pallas_call(
    _tc_y_body,
    out_shape=jax.ShapeDtypeStruct((_N, _H), jnp.float32),
)


def _tc_xu_body(x_ref, wupd_ref, xu_ref):
    xu_ref[...] = jnp.dot(x_ref[...], wupd_ref[: _D])


_tc_xu = pl.pallas_call(
    _tc_xu_body,
    out_shape=jax.ShapeDtypeStruct((_N, _H), jnp.float32),
)


_sc_mesh = plsc.VectorSubcoreMesh(core_axis_name="c", subcore_axis_name="s")


_NI = 4


@functools.partial(
    pl.kernel,
    mesh=_sc_mesh,
    out_type=jax.ShapeDtypeStruct((_NC, _NP, _H), jnp.float32),
    scratch_types=[
        [pltpu.VMEM((2, _CH), jnp.int32) for _ in range(_NI)],
        [pltpu.VMEM((_CH, _H), jnp.float32) for _ in range(_NB)],
        pltpu.VMEM((_ZR, _H), jnp.float32),
        pltpu.VMEM_SHARED((_NP, _H), jnp.float32),
        pltpu.SemaphoreType.DMA,
        pltpu.SemaphoreType.DMA,
    ],
)
def _sc_agg(y_hbm, ei_hbm, out_hbm,
            idxb, rows, zbuf, agg_sh, gsem, isem):
    c = lax.axis_index("c")
    s = lax.axis_index("s")
    t = c * _NS + s
    ebase = t * _EPT
    xbase = _TILES * _EPT + t * _CH

    def load_idx(off, k):
        pltpu.async_copy(ei_hbm.at[pl.ds(0, 2), pl.ds(off, _CH)],
                         idxb[k], isem)

    def drain_idx(k):
        pltpu.make_async_copy(
            ei_hbm.at[pl.ds(0, 2), pl.ds(0, _CH)], idxb[k], isem).wait()

    for k in range(_NI):
        load_idx(ebase + k * _CH, k)
    for b in range(_NB):
        drain_idx(b)
        pltpu.async_copy(y_hbm.at[idxb[b].at[0]], rows[b], gsem)

    zero16 = jnp.zeros((16,), jnp.float32)
    for r in range(_ZR):
        for col in range(_H // 16):
            zbuf[r, pl.ds(col * 16, 16)] = zero16
    row0 = s * _RPT
    for j in range(_RPT // _ZR):
        pltpu.sync_copy(zbuf, agg_sh.at[pl.ds(row0 + j * _ZR, _ZR)])
    plsc.subcore_barrier()

    def step(i, iv, load, feed):
        b = i % _NB
        k = i % _NI
        pltpu.make_async_copy(
            y_hbm.at[idxb[k].at[0]], rows[b], gsem).wait()
        pltpu.sync_copy(rows[b], agg_sh.at[idxb[k].at[1]], add=True)
        if load:
            load_idx(ebase + (iv + _NI) * _CH, k)
        if feed:
            kg = (i + _NB) % _NI
            drain_idx(kg)
            pltpu.async_copy(y_hbm.at[idxb[kg].at[0]], rows[b], gsem)

    _MAIN = (_CPT - _NI - 2) // _NI * _NI

    def body(j, carry):
        for u in range(_NI):
            step(u, j * _NI + u, True, True)
        return carry

    lax.fori_loop(0, _MAIN // _NI, body, 0)
    for i in range(_MAIN, _CPT):
        step(i, i, i + _NI < _CPT, i + _NB < _CPT)

    @pl.when(t < _XTRA)
    def _():
        load_idx(xbase, 0)
        drain_idx(0)
        pltpu.async_copy(y_hbm.at[idxb[0].at[0]], rows[0], gsem).wait()
        pltpu.sync_copy(rows[0], agg_sh.at[idxb[0].at[1]], add=True)

    plsc.subcore_barrier()
    pltpu.sync_copy(agg_sh.at[pl.ds(row0, _RPT)],
                    out_hbm.at[c, pl.ds(row0, _RPT)])


def _tc_post_body(xu_ref, aggs_ref, wupd_ref, w1_ref, w2t_ref, out_ref):
    agg = aggs_ref[0, : _N] + aggs_ref[1, : _N]
    nf = jnp.maximum(xu_ref[...] + jnp.dot(agg, wupd_ref[_D:]), 0.0)
    h1 = jnp.maximum(jnp.dot(nf, w1_ref[...]), 0.0)
    logits = jnp.sum(h1 * w2t_ref[...], axis=1, keepdims=True)
    m = jnp.max(logits)
    e = jnp.exp(logits - m)
    out_ref[...] = e / jnp.sum(e)


_tc_post = pl.pallas_call(
    _tc_post_body,
    out_shape=jax.ShapeDtypeStruct((_N, 1), jnp.float32),
)


def kernel(x, edge_index, W_msg, W_upd, w_actor1, w_actor2):
    y = _tc_y(x, W_msg)
    xu = _tc_xu(x, W_upd)
    aggs = _sc_agg(y, edge_index)
    probs = _tc_post(xu, aggs, W_upd, w_actor1, w_actor2.T)
    return probs[:, 0]
